# baseline (speedup 1.0000x reference)
import functools

import jax
import jax.numpy as jnp
from jax import lax
from jax.experimental import pallas as pl
from jax.experimental.pallas import tpu as pltpu
from jax.experimental.pallas import tpu_sc as plsc

HIDDEN = 1024
BATCH = 4
SEQ = 4096
B = BATCH * SEQ
NW = 32
B_PER_W = B // NW
CHUNK = 32
NCHUNK = B_PER_W // CHUNK
NBUF = 2
NSTEP = NCHUNK // NBUF

_mesh = plsc.VectorSubcoreMesh(core_axis_name="c", subcore_axis_name="s")


@functools.partial(
    pl.kernel,
    mesh=_mesh,
    out_type=jax.ShapeDtypeStruct((B, HIDDEN), jnp.float32),
    scratch_types=[
        pltpu.VMEM((B_PER_W,), jnp.int32),
        [pltpu.VMEM((CHUNK, HIDDEN), jnp.float32) for _ in range(NBUF)],
        [pltpu.SemaphoreType.DMA for _ in range(NBUF)],
        [pltpu.SemaphoreType.DMA for _ in range(NBUF)],
    ],
)
def _emb_lookup(table_hbm, idx_hbm, out_hbm, idx_v, bufs, gsems, osems):
    wid = lax.axis_index("s") * 2 + lax.axis_index("c")
    base = wid * B_PER_W
    row = wid // (SEQ // B_PER_W)
    col = (wid % (SEQ // B_PER_W)) * B_PER_W
    pltpu.sync_copy(idx_hbm.at[row, pl.ds(col, B_PER_W)], idx_v)

    def gather(i, b):
        return pltpu.async_copy(
            table_hbm.at[idx_v.at[pl.ds(i * CHUNK, CHUNK)]],
            bufs[b],
            gsems[b],
        )

    def scatter(i, b):
        return pltpu.async_copy(
            bufs[b],
            out_hbm.at[pl.ds(base + i * CHUNK, CHUNK)],
            osems[b],
        )

    def gwait(b):
        pltpu.make_async_copy(
            table_hbm.at[pl.ds(0, CHUNK)], bufs[b], gsems[b]
        ).wait()

    def owait(b):
        pltpu.make_async_copy(
            bufs[b], out_hbm.at[pl.ds(base, CHUNK)], osems[b]
        ).wait()

    for b in range(NBUF):
        gather(b, b)

    @pl.loop(0, NSTEP - 1)
    def _round(j):
        i0 = j * NBUF
        for b in range(NBUF):
            gwait(b)
            scatter(i0 + b, b)
            owait(b)
            gather(i0 + NBUF + b, b)

    i0 = (NSTEP - 1) * NBUF
    for b in range(NBUF):
        gwait(b)
        scatter(i0 + b, b)
    for b in range(NBUF):
        owait(b)


def kernel(input_ids, embedding):
    ids = input_ids.astype(jnp.int32)
    out = _emb_lookup(embedding, ids)
    return out.reshape(BATCH, SEQ, HIDDEN)


# speedup vs baseline: 1.0095x; 1.0095x over previous
import functools

import jax
import jax.numpy as jnp
from jax import lax
from jax.experimental import pallas as pl
from jax.experimental.pallas import tpu as pltpu
from jax.experimental.pallas import tpu_sc as plsc

HIDDEN = 1024
BATCH = 4
SEQ = 4096
B = BATCH * SEQ
NW = 32
B_PER_W = B // NW
CHUNK = 16
NCHUNK = B_PER_W // CHUNK
NBUF = 4
NSTEP = NCHUNK // NBUF

_mesh = plsc.VectorSubcoreMesh(core_axis_name="c", subcore_axis_name="s")


@functools.partial(
    pl.kernel,
    mesh=_mesh,
    out_type=jax.ShapeDtypeStruct((B, HIDDEN), jnp.float32),
    scratch_types=[
        pltpu.VMEM((B_PER_W,), jnp.int32),
        [pltpu.VMEM((CHUNK, HIDDEN), jnp.float32) for _ in range(NBUF)],
        [pltpu.SemaphoreType.DMA for _ in range(NBUF)],
        [pltpu.SemaphoreType.DMA for _ in range(NBUF)],
    ],
)
def _emb_lookup(table_hbm, idx_hbm, out_hbm, idx_v, bufs, gsems, osems):
    wid = lax.axis_index("s") * 2 + lax.axis_index("c")
    base = wid * B_PER_W
    row = wid // (SEQ // B_PER_W)
    col = (wid % (SEQ // B_PER_W)) * B_PER_W
    pltpu.sync_copy(idx_hbm.at[row, pl.ds(col, B_PER_W)], idx_v)

    def gather(i, b):
        return pltpu.async_copy(
            table_hbm.at[idx_v.at[pl.ds(i * CHUNK, CHUNK)]],
            bufs[b],
            gsems[b],
        )

    def scatter(i, b):
        return pltpu.async_copy(
            bufs[b],
            out_hbm.at[pl.ds(base + i * CHUNK, CHUNK)],
            osems[b],
        )

    def gwait(b):
        pltpu.make_async_copy(
            table_hbm.at[pl.ds(0, CHUNK)], bufs[b], gsems[b]
        ).wait()

    def owait(b):
        pltpu.make_async_copy(
            bufs[b], out_hbm.at[pl.ds(base, CHUNK)], osems[b]
        ).wait()

    for b in range(NBUF):
        gather(b, b)

    @pl.loop(0, NSTEP - 1)
    def _round(j):
        i0 = j * NBUF
        for b in range(NBUF):
            gwait(b)
            scatter(i0 + b, b)
            owait(b)
            gather(i0 + NBUF + b, b)

    i0 = (NSTEP - 1) * NBUF
    for b in range(NBUF):
        gwait(b)
        scatter(i0 + b, b)
    for b in range(NBUF):
        owait(b)


def kernel(input_ids, embedding):
    ids = input_ids.astype(jnp.int32)
    out = _emb_lookup(embedding, ids)
    return out.reshape(BATCH, SEQ, HIDDEN)
